# baseline (speedup 1.0000x reference)
import functools

import jax
import jax.numpy as jnp
from jax import lax
from jax.experimental import pallas as pl
from jax.experimental.pallas import tpu as pltpu
from jax.experimental.pallas import tpu_sc as plsc

_D = 1024
_SCALE = float(_D ** 0.5)
_LANES = 16

_NC = 2
_NS = 16
_NW = _NC * _NS


def _sc_body(n_tok, tok_per_w, chunk, idx_hbm, src_hbm, table_hbm, out_hbm,
             idx_v, rows_v, src_v, gsem, ssem):
    wid = lax.axis_index("s") * _NC + lax.axis_index("c")
    base = wid * tok_per_w
    n_chunks = tok_per_w // chunk

    pltpu.sync_copy(idx_hbm.at[pl.ds(base, tok_per_w)], idx_v)

    def chunk_body(c, carry):
        coff = pl.multiple_of(c * chunk, 8)
        off = pl.multiple_of(base + c * chunk, 8)
        g = pltpu.async_copy(table_hbm.at[idx_v.at[pl.ds(coff, chunk)]],
                             rows_v, gsem)
        s = pltpu.async_copy(src_hbm.at[pl.ds(off, chunk)], src_v, ssem)
        s.wait()
        g.wait()

        def row_body(r, rcarry):
            for j in range(_D // _LANES):
                sl = pl.ds(j * _LANES, _LANES)
                rows_v[r, sl] = src_v[r, sl] * _SCALE + rows_v[r, sl]
            return rcarry

        lax.fori_loop(0, chunk, row_body, 0)
        pltpu.sync_copy(rows_v, out_hbm.at[pl.ds(off, chunk)])
        return carry

    lax.fori_loop(0, n_chunks, chunk_body, 0)


@functools.partial(jax.jit, static_argnames=("n_tok", "chunk"))
def _sc_call(idx, src, table, n_tok, chunk):
    tok_per_w = n_tok // _NW
    mesh = plsc.VectorSubcoreMesh(core_axis_name="c", subcore_axis_name="s")
    body = functools.partial(_sc_body, n_tok, tok_per_w, chunk)
    return pl.kernel(
        body,
        out_type=jax.ShapeDtypeStruct((n_tok, _D), jnp.float32),
        mesh=mesh,
        scratch_types=[
            pltpu.VMEM((tok_per_w,), jnp.int32),
            pltpu.VMEM((chunk, _D), jnp.float32),
            pltpu.VMEM((chunk, _D), jnp.float32),
            pltpu.SemaphoreType.DMA,
            pltpu.SemaphoreType.DMA,
        ],
    )(idx, src, table)


def kernel(src_word, src_pos, pos_table):
    b, s, d = src_word.shape
    n_tok = b * s
    src = src_word.reshape(n_tok, d)
    idx = src_pos.reshape(n_tok)
    out = _sc_call(idx, src, pos_table, n_tok, 32)
    return out.reshape(b, s, d)


# speedup vs baseline: 1.3676x; 1.3676x over previous
import functools

import jax
import jax.numpy as jnp
from jax import lax
from jax.experimental import pallas as pl
from jax.experimental.pallas import tpu as pltpu
from jax.experimental.pallas import tpu_sc as plsc

_D = 1024
_SCALE = float(_D ** 0.5)
_LANES = 16

_NC = 2
_NS = 16
_NW = _NC * _NS
_NBUF = 3


def _sc_body(tok_per_w, chunk, idx_hbm, src_hbm, table_hbm, out_hbm,
             idx_v, *bufs):
    rows = bufs[0:_NBUF]
    src = bufs[_NBUF:2 * _NBUF]
    gsem = bufs[2 * _NBUF:3 * _NBUF]
    ssem = bufs[3 * _NBUF:4 * _NBUF]
    osem = bufs[4 * _NBUF:5 * _NBUF]

    wid = lax.axis_index("s") * _NC + lax.axis_index("c")
    base = wid * tok_per_w
    n_chunks = tok_per_w // chunk

    pltpu.sync_copy(idx_hbm.at[pl.ds(base, tok_per_w)], idx_v)

    def issue_in(c, b):
        g = pltpu.async_copy(table_hbm.at[idx_v.at[pl.ds(c * chunk, chunk)]],
                             rows[b], gsem[b])
        s = pltpu.async_copy(src_hbm.at[pl.ds(base + c * chunk, chunk)],
                             src[b], ssem[b])
        return g, s

    in_flight = {}
    out_flight = {}
    for c in range(min(_NBUF, n_chunks)):
        in_flight[c] = issue_in(c, c % _NBUF)

    for c in range(n_chunks):
        b = c % _NBUF
        g, s = in_flight.pop(c)
        g.wait()
        s.wait()

        def row_body(r, rcarry):
            for j in range(_D // _LANES):
                sl = pl.ds(j * _LANES, _LANES)
                plsc.addupdate(rows[b].at[r, sl], src[b][r, sl] * _SCALE)
            return rcarry

        lax.fori_loop(0, chunk, row_body, 0)

        out_flight[c] = pltpu.async_copy(
            rows[b], out_hbm.at[pl.ds(base + c * chunk, chunk)], osem[b])

        prev = c - 1
        nxt = prev + _NBUF
        if prev >= 0 and nxt < n_chunks:
            out_flight.pop(prev).wait()
            in_flight[nxt] = issue_in(nxt, prev % _NBUF)

    for c in sorted(out_flight):
        out_flight.pop(c).wait()


@functools.partial(jax.jit, static_argnames=("n_tok", "chunk"))
def _sc_call(idx, src, table, n_tok, chunk):
    tok_per_w = n_tok // _NW
    mesh = plsc.VectorSubcoreMesh(core_axis_name="c", subcore_axis_name="s")
    body = functools.partial(_sc_body, tok_per_w, chunk)
    return pl.kernel(
        body,
        out_type=jax.ShapeDtypeStruct((n_tok, _D), jnp.float32),
        mesh=mesh,
        scratch_types=(
            [pltpu.VMEM((tok_per_w,), jnp.int32)]
            + [pltpu.VMEM((chunk, _D), jnp.float32) for _ in range(2 * _NBUF)]
            + [pltpu.SemaphoreType.DMA for _ in range(3 * _NBUF)]
        ),
    )(idx, src, table)


def kernel(src_word, src_pos, pos_table):
    b, s, d = src_word.shape
    n_tok = b * s
    src = src_word.reshape(n_tok, d)
    idx = src_pos.reshape(n_tok)
    out = _sc_call(idx, src, pos_table, n_tok, 16)
    return out.reshape(b, s, d)


# speedup vs baseline: 1.3859x; 1.0134x over previous
import functools

import jax
import jax.numpy as jnp
from jax import lax
from jax.experimental import pallas as pl
from jax.experimental.pallas import tpu as pltpu
from jax.experimental.pallas import tpu_sc as plsc

_D = 1024
_SCALE = float(_D ** 0.5)
_LANES = 16

_NC = 2
_NS = 16
_NW = _NC * _NS
_NR = 4
_NSRC = 3
_PREF = 3


def _sc_body(tok_per_w, chunk, idx_hbm, src_hbm, table_hbm, out_hbm,
             idx_v, *bufs):
    rows = bufs[0:_NR]
    src = bufs[_NR:_NR + _NSRC]
    o = _NR + _NSRC
    gsem = bufs[o:o + _NR]
    ssem = bufs[o + _NR:o + _NR + _NSRC]
    osem = bufs[o + _NR + _NSRC:o + 2 * _NR + _NSRC]

    wid = lax.axis_index("s") * _NC + lax.axis_index("c")
    base = wid * tok_per_w
    n_chunks = tok_per_w // chunk

    pltpu.sync_copy(idx_hbm.at[pl.ds(base, tok_per_w)], idx_v)

    def issue_in(c):
        rb, sb = c % _NR, c % _NSRC
        g = pltpu.async_copy(table_hbm.at[idx_v.at[pl.ds(c * chunk, chunk)]],
                             rows[rb], gsem[rb])
        s = pltpu.async_copy(src_hbm.at[pl.ds(base + c * chunk, chunk)],
                             src[sb], ssem[sb])
        return g, s

    in_flight = {}
    out_flight = {}
    for c in range(min(_PREF, n_chunks)):
        in_flight[c] = issue_in(c)

    for c in range(n_chunks):
        rb, sb = c % _NR, c % _NSRC
        g, s = in_flight.pop(c)
        g.wait()
        s.wait()

        def row_body(r, rcarry):
            for j in range(_D // _LANES):
                sl = pl.ds(j * _LANES, _LANES)
                plsc.addupdate(rows[rb].at[r, sl], src[sb][r, sl] * _SCALE)
            return rcarry

        lax.fori_loop(0, chunk, row_body, 0)

        out_flight[c] = pltpu.async_copy(
            rows[rb], out_hbm.at[pl.ds(base + c * chunk, chunk)], osem[rb])

        nxt = c + _PREF
        if nxt < n_chunks:
            old = nxt - _NR
            if old >= 0:
                out_flight.pop(old).wait()
            in_flight[nxt] = issue_in(nxt)

    for c in sorted(out_flight):
        out_flight.pop(c).wait()


@functools.partial(jax.jit, static_argnames=("n_tok", "chunk"))
def _sc_call(idx, src, table, n_tok, chunk):
    tok_per_w = n_tok // _NW
    mesh = plsc.VectorSubcoreMesh(core_axis_name="c", subcore_axis_name="s")
    body = functools.partial(_sc_body, tok_per_w, chunk)
    return pl.kernel(
        body,
        out_type=jax.ShapeDtypeStruct((n_tok, _D), jnp.float32),
        mesh=mesh,
        scratch_types=(
            [pltpu.VMEM((tok_per_w,), jnp.int32)]
            + [pltpu.VMEM((chunk, _D), jnp.float32)
               for _ in range(_NR + _NSRC)]
            + [pltpu.SemaphoreType.DMA for _ in range(2 * _NR + _NSRC)]
        ),
    )(idx, src, table)


def kernel(src_word, src_pos, pos_table):
    b, s, d = src_word.shape
    n_tok = b * s
    src = src_word.reshape(n_tok, d)
    idx = src_pos.reshape(n_tok)
    out = _sc_call(idx, src, pos_table, n_tok, 16)
    return out.reshape(b, s, d)
